# baseline (speedup 1.0000x reference)
import jax
import jax.numpy as jnp
from jax import lax
from jax.experimental import pallas as pl
from jax.experimental.pallas import tpu as pltpu
from jax.experimental.pallas import tpu_sc as plsc

VOCAB = 100000
HID = 2048
EPS = 1e-12
L = 16
NW = 32
N_TOK = 4 * 4096
PER_W = N_TOK // NW
CK = 16
NCHUNK = PER_W // CK
NSLICE = HID // L


def _bcast_lane(v, lane, iota16):
    idx = (iota16 & 0) + lane
    return jnp.take_along_axis(v, idx, axis=0,
                               mode=lax.GatherScatterMode.PROMISE_IN_BOUNDS)


def _allsum(v, iota16):
    for step in (1, 2, 4, 8):
        pv = iota16 ^ step
        v = v + jnp.take_along_axis(
            v, pv, axis=0, mode=lax.GatherScatterMode.PROMISE_IN_BOUNDS)
    return v


def _rsqrt16(x, iota16):
    i = lax.bitcast_convert_type(x, jnp.int32)
    magic = (iota16 & 0) + 0x5F3759DF
    y = lax.bitcast_convert_type(magic - (i >> 1), jnp.float32)
    for _ in range(3):
        y = y * (1.5 - 0.5 * x * y * y)
    return y


def _body(ids_hbm, table_hbm, gamma_hbm, beta_hbm, out_hbm,
          ids_v, rows_v, gamma_v, beta_v, sem):
    cid = lax.axis_index("c")
    sid = lax.axis_index("s")
    wid = sid * 2 + cid
    base = wid * PER_W
    iota16 = lax.iota(jnp.int32, L)

    pltpu.sync_copy(ids_hbm.at[pl.ds(base, PER_W)], ids_v)
    pltpu.sync_copy(gamma_hbm, gamma_v)
    pltpu.sync_copy(beta_hbm, beta_v)

    def chunk_body(c, carry):
        row0 = c * CK
        pltpu.async_copy(table_hbm.at[ids_v.at[pl.ds(row0, CK)]],
                         rows_v, sem).wait()
        ids_chunk = ids_v[pl.ds(row0, CK)]
        scale_chunk = jnp.where(ids_chunk == 0, 0.0, 1.0)
        for r in range(CK):
            scale = _bcast_lane(scale_chunk, r, iota16)

            def p1(j, sc):
                s, s2 = sc
                v = rows_v[r, pl.ds(j * L, L)]
                return s + v, s2 + v * v

            zz = iota16.astype(jnp.float32) * 0.0
            s, s2 = lax.fori_loop(0, NSLICE, p1, (zz, zz))
            mean_v = _allsum(s, iota16) * (scale * (1.0 / HID))
            var_v = (_allsum(s2, iota16) * (scale * (1.0 / HID))
                     - mean_v * mean_v)
            var_v = jnp.maximum(var_v, 0.0) + EPS
            inv_v = _rsqrt16(var_v, iota16) * scale

            def p2(j, _):
                off = j * L
                v = rows_v[r, pl.ds(off, L)]
                g = gamma_v[pl.ds(off, L)]
                b = beta_v[pl.ds(off, L)]
                rows_v[r, pl.ds(off, L)] = (v - mean_v) * (inv_v * g) + b
                return 0

            lax.fori_loop(0, NSLICE, p2, 0)

        pltpu.sync_copy(rows_v, out_hbm.at[pl.ds(base + row0, CK)])
        return carry

    lax.fori_loop(0, NCHUNK, chunk_body, 0)


def kernel(token_ids, position_ids, table, gamma, beta):
    del position_ids
    B, S = token_ids.shape
    ids = token_ids.reshape(N_TOK).astype(jnp.int32)

    mesh = plsc.VectorSubcoreMesh(core_axis_name="c", subcore_axis_name="s")
    out = pl.kernel(
        _body,
        out_type=jax.ShapeDtypeStruct((N_TOK, HID), jnp.float32),
        mesh=mesh,
        scratch_types=[
            pltpu.VMEM((PER_W,), jnp.int32),
            pltpu.VMEM((CK, HID), jnp.float32),
            pltpu.VMEM((HID,), jnp.float32),
            pltpu.VMEM((HID,), jnp.float32),
            pltpu.SemaphoreType.DMA,
        ],
    )(ids, table, gamma, beta)
    return out.reshape(B, S, HID)


# speedup vs baseline: 1.2435x; 1.2435x over previous
import jax
import jax.numpy as jnp
from jax import lax
from jax.experimental import pallas as pl
from jax.experimental.pallas import tpu as pltpu
from jax.experimental.pallas import tpu_sc as plsc

VOCAB = 100000
HID = 2048
EPS = 1e-12
L = 16
NW = 32
N_TOK = 4 * 4096
PER_W = N_TOK // NW
CK = 16
NCHUNK = PER_W // CK
NSLICE = HID // L


def _bcast_lane(v, lane, iota16):
    idx = (iota16 & 0) + lane
    return jnp.take_along_axis(v, idx, axis=0,
                               mode=lax.GatherScatterMode.PROMISE_IN_BOUNDS)


def _allsum(v, iota16):
    for step in (1, 2, 4, 8):
        pv = iota16 ^ step
        v = v + jnp.take_along_axis(
            v, pv, axis=0, mode=lax.GatherScatterMode.PROMISE_IN_BOUNDS)
    return v


def _rsqrt16(x, iota16):
    i = lax.bitcast_convert_type(x, jnp.int32)
    magic = (iota16 & 0) + 0x5F3759DF
    y = lax.bitcast_convert_type(magic - (i >> 1), jnp.float32)
    for _ in range(3):
        y = y * (1.5 - 0.5 * x * y * y)
    return y


def _body(ids_hbm, table_hbm, gamma_hbm, beta_hbm, out_hbm,
          ids_v, rows_v, gamma_v, beta_v, sem):
    cid = lax.axis_index("c")
    sid = lax.axis_index("s")
    wid = sid * 2 + cid
    base = wid * PER_W
    iota16 = lax.iota(jnp.int32, L)

    pltpu.sync_copy(ids_hbm.at[pl.ds(base, PER_W)], ids_v)
    pltpu.sync_copy(gamma_hbm, gamma_v)
    pltpu.sync_copy(beta_hbm, beta_v)

    def chunk_body(c, carry):
        row0 = c * CK
        pltpu.async_copy(table_hbm.at[ids_v.at[pl.ds(row0, CK)]],
                         rows_v, sem).wait()
        ids_chunk = ids_v[pl.ds(row0, CK)]
        scale_chunk = jnp.where(ids_chunk == 0, 0.0, 1.0)
        for r in range(CK):
            scale = _bcast_lane(scale_chunk, r, iota16)

            def p1(j, sc):
                s, s2 = sc
                v = rows_v[r, pl.ds(j * L, L)]
                return s + v, s2 + v * v

            zz = iota16.astype(jnp.float32) * 0.0
            s, s2 = lax.fori_loop(0, NSLICE, p1, (zz, zz), unroll=8)
            mean_v = _allsum(s, iota16) * (scale * (1.0 / HID))
            var_v = (_allsum(s2, iota16) * (scale * (1.0 / HID))
                     - mean_v * mean_v)
            var_v = jnp.maximum(var_v, 0.0) + EPS
            inv_v = _rsqrt16(var_v, iota16) * scale

            def p2(j, _):
                off = j * L
                v = rows_v[r, pl.ds(off, L)]
                g = gamma_v[pl.ds(off, L)]
                b = beta_v[pl.ds(off, L)]
                rows_v[r, pl.ds(off, L)] = (v - mean_v) * (inv_v * g) + b
                return 0

            lax.fori_loop(0, NSLICE, p2, 0, unroll=8)

        pltpu.sync_copy(rows_v, out_hbm.at[pl.ds(base + row0, CK)])
        return carry

    lax.fori_loop(0, NCHUNK, chunk_body, 0)


def kernel(token_ids, position_ids, table, gamma, beta):
    del position_ids
    B, S = token_ids.shape
    ids = token_ids.reshape(N_TOK).astype(jnp.int32)

    mesh = plsc.VectorSubcoreMesh(core_axis_name="c", subcore_axis_name="s")
    out = pl.kernel(
        _body,
        out_type=jax.ShapeDtypeStruct((N_TOK, HID), jnp.float32),
        mesh=mesh,
        scratch_types=[
            pltpu.VMEM((PER_W,), jnp.int32),
            pltpu.VMEM((CK, HID), jnp.float32),
            pltpu.VMEM((HID,), jnp.float32),
            pltpu.VMEM((HID,), jnp.float32),
            pltpu.SemaphoreType.DMA,
        ],
    )(ids, table, gamma, beta)
    return out.reshape(B, S, HID)


# speedup vs baseline: 1.5731x; 1.2650x over previous
import jax
import jax.numpy as jnp
from jax import lax
from jax.experimental import pallas as pl
from jax.experimental.pallas import tpu as pltpu
from jax.experimental.pallas import tpu_sc as plsc

VOCAB = 100000
HID = 2048
EPS = 1e-12
L = 16
NW = 32
N_TOK = 4 * 4096
PER_W = N_TOK // NW
CK = 16
NCHUNK = PER_W // CK
NSLICE = HID // L


def _bcast_lane(v, lane, iota16):
    idx = (iota16 & 0) + lane
    return jnp.take_along_axis(v, idx, axis=0,
                               mode=lax.GatherScatterMode.PROMISE_IN_BOUNDS)


def _allsum(v, iota16):
    for step in (1, 2, 4, 8):
        pv = iota16 ^ step
        v = v + jnp.take_along_axis(
            v, pv, axis=0, mode=lax.GatherScatterMode.PROMISE_IN_BOUNDS)
    return v


def _rsqrt16(x, iota16):
    i = lax.bitcast_convert_type(x, jnp.int32)
    magic = (iota16 & 0) + 0x5F3759DF
    y = lax.bitcast_convert_type(magic - (i >> 1), jnp.float32)
    for _ in range(3):
        y = y * (1.5 - 0.5 * x * y * y)
    return y


def _body(ids_hbm, table_hbm, gamma_hbm, beta_hbm, out_hbm,
          ids_v, rows_v, gamma_v, beta_v, sem):
    cid = lax.axis_index("c")
    sid = lax.axis_index("s")
    wid = sid * 2 + cid
    base = wid * PER_W
    iota16 = lax.iota(jnp.int32, L)

    pltpu.sync_copy(ids_hbm.at[pl.ds(base, PER_W)], ids_v)
    pltpu.sync_copy(gamma_hbm, gamma_v)
    pltpu.sync_copy(beta_hbm, beta_v)

    def chunk_body(c, carry):
        row0 = c * CK
        pltpu.async_copy(table_hbm.at[ids_v.at[pl.ds(row0, CK)]],
                         rows_v, sem).wait()
        ids_chunk = ids_v[pl.ds(row0, CK)]
        scale_chunk = jnp.where(ids_chunk == 0, 0.0, 1.0)
        zz = iota16.astype(jnp.float32) * 0.0

        means, invs = [], []
        for r in range(CK):
            scale = _bcast_lane(scale_chunk, r, iota16)

            def p1(j, sc):
                s, s2 = sc
                v = rows_v[r, pl.ds(j * L, L)]
                return s + v, s2 + v * v

            s, s2 = lax.fori_loop(0, NSLICE, p1, (zz, zz), unroll=8)
            mean_v = _allsum(s, iota16) * (scale * (1.0 / HID))
            var_v = (_allsum(s2, iota16) * (scale * (1.0 / HID))
                     - mean_v * mean_v)
            var_v = jnp.maximum(var_v, 0.0) + EPS
            means.append(mean_v)
            invs.append(_rsqrt16(var_v, iota16) * scale)

        def p2(j, _):
            off = j * L
            g = gamma_v[pl.ds(off, L)]
            b = beta_v[pl.ds(off, L)]
            for r in range(CK):
                v = rows_v[r, pl.ds(off, L)]
                rows_v[r, pl.ds(off, L)] = (v - means[r]) * (invs[r] * g) + b
            return 0

        lax.fori_loop(0, NSLICE, p2, 0, unroll=2)

        pltpu.sync_copy(rows_v, out_hbm.at[pl.ds(base + row0, CK)])
        return carry

    lax.fori_loop(0, NCHUNK, chunk_body, 0)


def kernel(token_ids, position_ids, table, gamma, beta):
    del position_ids
    B, S = token_ids.shape
    ids = token_ids.reshape(N_TOK).astype(jnp.int32)

    mesh = plsc.VectorSubcoreMesh(core_axis_name="c", subcore_axis_name="s")
    out = pl.kernel(
        _body,
        out_type=jax.ShapeDtypeStruct((N_TOK, HID), jnp.float32),
        mesh=mesh,
        scratch_types=[
            pltpu.VMEM((PER_W,), jnp.int32),
            pltpu.VMEM((CK, HID), jnp.float32),
            pltpu.VMEM((HID,), jnp.float32),
            pltpu.VMEM((HID,), jnp.float32),
            pltpu.SemaphoreType.DMA,
        ],
    )(ids, table, gamma, beta)
    return out.reshape(B, S, HID)


# speedup vs baseline: 1.7762x; 1.1291x over previous
import jax
import jax.numpy as jnp
from jax import lax
from jax.experimental import pallas as pl
from jax.experimental.pallas import tpu as pltpu
from jax.experimental.pallas import tpu_sc as plsc

VOCAB = 100000
HID = 2048
EPS = 1e-12
L = 16
NW = 32
N_TOK = 4 * 4096
PER_W = N_TOK // NW
CK = 16
NCHUNK = PER_W // CK
NSLICE = HID // L


def _bcast_lane(v, lane, iota16):
    idx = (iota16 & 0) + lane
    return jnp.take_along_axis(v, idx, axis=0,
                               mode=lax.GatherScatterMode.PROMISE_IN_BOUNDS)


def _allsum(v, iota16):
    for step in (1, 2, 4, 8):
        pv = iota16 ^ step
        v = v + jnp.take_along_axis(
            v, pv, axis=0, mode=lax.GatherScatterMode.PROMISE_IN_BOUNDS)
    return v


def _rsqrt16(x, iota16):
    i = lax.bitcast_convert_type(x, jnp.int32)
    magic = (iota16 & 0) + 0x5F3759DF
    y = lax.bitcast_convert_type(magic - (i >> 1), jnp.float32)
    for _ in range(3):
        y = y * (1.5 - 0.5 * x * y * y)
    return y


def _body(ids_hbm, table_hbm, gamma_hbm, beta_hbm, out_hbm,
          ids_v, rows2_v, gamma_v, beta_v, gsem0, gsem1, ssem0, ssem1):
    cid = lax.axis_index("c")
    sid = lax.axis_index("s")
    wid = sid * 2 + cid
    base = wid * PER_W
    iota16 = lax.iota(jnp.int32, L)

    pltpu.sync_copy(ids_hbm.at[pl.ds(base, PER_W)], ids_v)
    pltpu.sync_copy(gamma_hbm, gamma_v)
    pltpu.sync_copy(beta_hbm, beta_v)

    def g_copy(c, slot_ref, sem):
        return pltpu.make_async_copy(
            table_hbm.at[ids_v.at[pl.ds(c * CK, CK)]], slot_ref, sem)

    def s_copy(c, slot_ref, sem):
        return pltpu.make_async_copy(
            slot_ref, out_hbm.at[pl.ds(base + c * CK, CK)], sem)

    def compute(rows_v, c):
        ids_chunk = ids_v[pl.ds(c * CK, CK)]
        scale_chunk = jnp.where(ids_chunk == 0, 0.0, 1.0)
        zz = iota16.astype(jnp.float32) * 0.0

        means, invs = [], []
        for r in range(CK):
            scale = _bcast_lane(scale_chunk, r, iota16)

            def p1(j, sc):
                s, s2 = sc
                v = rows_v[r, pl.ds(j * L, L)]
                return s + v, s2 + v * v

            s, s2 = lax.fori_loop(0, NSLICE, p1, (zz, zz), unroll=8)
            mean_v = _allsum(s, iota16) * (scale * (1.0 / HID))
            var_v = (_allsum(s2, iota16) * (scale * (1.0 / HID))
                     - mean_v * mean_v)
            var_v = jnp.maximum(var_v, 0.0) + EPS
            means.append(mean_v)
            invs.append(_rsqrt16(var_v, iota16) * scale)

        def p2(j, _):
            off = j * L
            g = gamma_v[pl.ds(off, L)]
            b = beta_v[pl.ds(off, L)]
            for r in range(CK):
                v = rows_v[r, pl.ds(off, L)]
                rows_v[r, pl.ds(off, L)] = (v - means[r]) * (invs[r] * g) + b
            return 0

        lax.fori_loop(0, NSLICE, p2, 0, unroll=2)

    s0_ref = rows2_v.at[0]
    s1_ref = rows2_v.at[1]

    g_copy(0, s0_ref, gsem0).start()

    def pair_body(cc, carry):
        c0 = cc * 2

        @pl.when(cc > 0)
        def _():
            s_copy(c0 - 1, s1_ref, ssem1).wait()

        g_copy(c0 + 1, s1_ref, gsem1).start()
        g_copy(c0, s0_ref, gsem0).wait()
        compute(s0_ref, c0)
        s_copy(c0, s0_ref, ssem0).start()

        g_copy(c0 + 1, s1_ref, gsem1).wait()

        @pl.when(cc + 1 < NCHUNK // 2)
        def _():
            s_copy(c0, s0_ref, ssem0).wait()
            g_copy(c0 + 2, s0_ref, gsem0).start()

        compute(s1_ref, c0 + 1)
        s_copy(c0 + 1, s1_ref, ssem1).start()
        return carry

    lax.fori_loop(0, NCHUNK // 2, pair_body, 0)
    s_copy(NCHUNK - 2, s0_ref, ssem0).wait()
    s_copy(NCHUNK - 1, s1_ref, ssem1).wait()


def kernel(token_ids, position_ids, table, gamma, beta):
    del position_ids
    B, S = token_ids.shape
    ids = token_ids.reshape(N_TOK).astype(jnp.int32)

    mesh = plsc.VectorSubcoreMesh(core_axis_name="c", subcore_axis_name="s")
    out = pl.kernel(
        _body,
        out_type=jax.ShapeDtypeStruct((N_TOK, HID), jnp.float32),
        mesh=mesh,
        scratch_types=[
            pltpu.VMEM((PER_W,), jnp.int32),
            pltpu.VMEM((2, CK, HID), jnp.float32),
            pltpu.VMEM((HID,), jnp.float32),
            pltpu.VMEM((HID,), jnp.float32),
            pltpu.SemaphoreType.DMA,
            pltpu.SemaphoreType.DMA,
            pltpu.SemaphoreType.DMA,
            pltpu.SemaphoreType.DMA,
        ],
    )(ids, table, gamma, beta)
    return out.reshape(B, S, HID)


# speedup vs baseline: 1.8833x; 1.0603x over previous
import jax
import jax.numpy as jnp
from jax import lax
from jax.experimental import pallas as pl
from jax.experimental.pallas import tpu as pltpu
from jax.experimental.pallas import tpu_sc as plsc

VOCAB = 100000
HID = 2048
EPS = 1e-12
L = 16
NW = 32
N_TOK = 4 * 4096
PER_W = N_TOK // NW
CK = 16
NCHUNK = PER_W // CK
NSLICE = HID // L


def _bcast_lane(v, lane, iota16):
    idx = (iota16 & 0) + lane
    return jnp.take_along_axis(v, idx, axis=0,
                               mode=lax.GatherScatterMode.PROMISE_IN_BOUNDS)


def _allsum(v, iota16):
    for step in (1, 2, 4, 8):
        pv = iota16 ^ step
        v = v + jnp.take_along_axis(
            v, pv, axis=0, mode=lax.GatherScatterMode.PROMISE_IN_BOUNDS)
    return v


def _rsqrt16(x, iota16):
    i = lax.bitcast_convert_type(x, jnp.int32)
    magic = (iota16 & 0) + 0x5F3759DF
    y = lax.bitcast_convert_type(magic - (i >> 1), jnp.float32)
    for _ in range(3):
        y = y * (1.5 - 0.5 * x * y * y)
    return y


def _body(ids_hbm, table_hbm, gamma_hbm, beta_hbm, out_hbm,
          ids_v, rows2_v, gamma_v, beta_v, gsem0, gsem1, ssem0, ssem1):
    cid = lax.axis_index("c")
    sid = lax.axis_index("s")
    wid = sid * 2 + cid
    base = wid * PER_W
    iota16 = lax.iota(jnp.int32, L)

    pltpu.sync_copy(ids_hbm.at[pl.ds(base, PER_W)], ids_v)
    pltpu.sync_copy(gamma_hbm, gamma_v)
    pltpu.sync_copy(beta_hbm, beta_v)

    def g_copy(c, slot_ref, sem):
        return pltpu.make_async_copy(
            table_hbm.at[ids_v.at[pl.ds(c * CK, CK)]], slot_ref, sem)

    def s_copy(c, slot_ref, sem):
        return pltpu.make_async_copy(
            slot_ref, out_hbm.at[pl.ds(base + c * CK, CK)], sem)

    def compute(rows_v, c):
        ids_chunk = ids_v[pl.ds(c * CK, CK)]
        scale_chunk = jnp.where(ids_chunk == 0, 0.0, 1.0)
        zz = iota16.astype(jnp.float32) * 0.0

        means, invs = [], []
        for r in range(CK):
            scale = _bcast_lane(scale_chunk, r, iota16)

            def p1(j, accs):
                accs = list(accs)
                off = j * (8 * L)
                for k in range(8):
                    v = rows_v[r, pl.ds(off + k * L, L)]
                    accs[k] = accs[k] + v
                    accs[8 + k] = accs[8 + k] + v * v
                return tuple(accs)

            accs = lax.fori_loop(0, NSLICE // 8, p1, (zz,) * 16, unroll=2)
            s = (((accs[0] + accs[1]) + (accs[2] + accs[3]))
                 + ((accs[4] + accs[5]) + (accs[6] + accs[7])))
            s2 = (((accs[8] + accs[9]) + (accs[10] + accs[11]))
                  + ((accs[12] + accs[13]) + (accs[14] + accs[15])))
            mean_v = _allsum(s, iota16) * (scale * (1.0 / HID))
            var_v = (_allsum(s2, iota16) * (scale * (1.0 / HID))
                     - mean_v * mean_v)
            var_v = jnp.maximum(var_v, 0.0) + EPS
            means.append(mean_v)
            invs.append(_rsqrt16(var_v, iota16) * scale)

        def p2(j, _):
            off = j * L
            g = gamma_v[pl.ds(off, L)]
            b = beta_v[pl.ds(off, L)]
            for r in range(CK):
                v = rows_v[r, pl.ds(off, L)]
                rows_v[r, pl.ds(off, L)] = (v - means[r]) * (invs[r] * g) + b
            return 0

        lax.fori_loop(0, NSLICE, p2, 0, unroll=2)

    s0_ref = rows2_v.at[0]
    s1_ref = rows2_v.at[1]

    g_copy(0, s0_ref, gsem0).start()

    def pair_body(cc, carry):
        c0 = cc * 2

        @pl.when(cc > 0)
        def _():
            s_copy(c0 - 1, s1_ref, ssem1).wait()

        g_copy(c0 + 1, s1_ref, gsem1).start()
        g_copy(c0, s0_ref, gsem0).wait()
        compute(s0_ref, c0)
        s_copy(c0, s0_ref, ssem0).start()

        g_copy(c0 + 1, s1_ref, gsem1).wait()

        @pl.when(cc + 1 < NCHUNK // 2)
        def _():
            s_copy(c0, s0_ref, ssem0).wait()
            g_copy(c0 + 2, s0_ref, gsem0).start()

        compute(s1_ref, c0 + 1)
        s_copy(c0 + 1, s1_ref, ssem1).start()
        return carry

    lax.fori_loop(0, NCHUNK // 2, pair_body, 0)
    s_copy(NCHUNK - 2, s0_ref, ssem0).wait()
    s_copy(NCHUNK - 1, s1_ref, ssem1).wait()


def kernel(token_ids, position_ids, table, gamma, beta):
    del position_ids
    B, S = token_ids.shape
    ids = token_ids.reshape(N_TOK).astype(jnp.int32)

    mesh = plsc.VectorSubcoreMesh(core_axis_name="c", subcore_axis_name="s")
    out = pl.kernel(
        _body,
        out_type=jax.ShapeDtypeStruct((N_TOK, HID), jnp.float32),
        mesh=mesh,
        scratch_types=[
            pltpu.VMEM((PER_W,), jnp.int32),
            pltpu.VMEM((2, CK, HID), jnp.float32),
            pltpu.VMEM((HID,), jnp.float32),
            pltpu.VMEM((HID,), jnp.float32),
            pltpu.SemaphoreType.DMA,
            pltpu.SemaphoreType.DMA,
            pltpu.SemaphoreType.DMA,
            pltpu.SemaphoreType.DMA,
        ],
    )(ids, table, gamma, beta)
    return out.reshape(B, S, HID)
